# baseline (speedup 1.0000x reference)
import functools

import jax
import jax.numpy as jnp
from jax import lax
from jax.experimental import pallas as pl
from jax.experimental.pallas import tpu as pltpu
from jax.experimental.pallas import tpu_sc as plsc

_BANK = 100000
_DIM = 128
_BATCH = 16384

_CHUNK = 4000
_NCHUNK = _BANK // _CHUNK
_FSUB = 800
_NSUB = _CHUNK // _FSUB
_WSPLIT = _BATCH // _CHUNK
_SOFF = _BATCH - _WSPLIT * _CHUNK


def _body(feat_hbm, bank_hbm, lab_hbm, lbank_hbm, out_fb, out_lb, fbuf, lbuf):
    wid = lax.axis_index("s") * 2 + lax.axis_index("c")
    base = wid * _CHUNK

    def _fb_sub(src, j):
        row0 = base + j * _FSUB
        pltpu.sync_copy(src.at[pl.ds(row0, _FSUB), :], fbuf)
        pltpu.sync_copy(fbuf, out_fb.at[pl.ds(row0, _FSUB), :])

    @pl.when(wid < _WSPLIT)
    def _():
        for j in range(_NSUB):
            _fb_sub(feat_hbm, j)
        pltpu.sync_copy(lab_hbm.at[pl.ds(base, _CHUNK)], lbuf)
        pltpu.sync_copy(lbuf, out_lb.at[pl.ds(base, _CHUNK)])

    @pl.when(wid == _WSPLIT)
    def _():
        pltpu.sync_copy(feat_hbm.at[pl.ds(base, _SOFF), :],
                        fbuf.at[pl.ds(0, _SOFF), :])
        pltpu.sync_copy(bank_hbm.at[pl.ds(base + _SOFF, _FSUB - _SOFF), :],
                        fbuf.at[pl.ds(_SOFF, _FSUB - _SOFF), :])
        pltpu.sync_copy(fbuf, out_fb.at[pl.ds(base, _FSUB), :])
        for j in range(1, _NSUB):
            _fb_sub(bank_hbm, j)
        pltpu.sync_copy(lab_hbm.at[pl.ds(base, _SOFF)], lbuf.at[pl.ds(0, _SOFF)])
        pltpu.sync_copy(lbank_hbm.at[pl.ds(base + _SOFF, _CHUNK - _SOFF)],
                        lbuf.at[pl.ds(_SOFF, _CHUNK - _SOFF)])
        pltpu.sync_copy(lbuf, out_lb.at[pl.ds(base, _CHUNK)])

    @pl.when(jnp.logical_and(wid > _WSPLIT, wid < _NCHUNK))
    def _():
        for j in range(_NSUB):
            _fb_sub(bank_hbm, j)
        pltpu.sync_copy(lbank_hbm.at[pl.ds(base, _CHUNK)], lbuf)
        pltpu.sync_copy(lbuf, out_lb.at[pl.ds(base, _CHUNK)])


def kernel(features, labels, feature_bank, label_bank):
    run = functools.partial(
        pl.kernel,
        out_type=[
            jax.ShapeDtypeStruct((_BANK, _DIM), feature_bank.dtype),
            jax.ShapeDtypeStruct((_BANK,), label_bank.dtype),
        ],
        mesh=plsc.VectorSubcoreMesh(core_axis_name="c", subcore_axis_name="s"),
        scratch_types=[
            pltpu.VMEM((_FSUB, _DIM), feature_bank.dtype),
            pltpu.VMEM((_CHUNK,), label_bank.dtype),
        ],
    )(_body)
    out_fb, out_lb = run(features, feature_bank, labels, label_bank)
    return out_fb, out_lb


# speedup vs baseline: 1.0892x; 1.0892x over previous
import functools

import jax
import jax.numpy as jnp
from jax import lax
from jax.experimental import pallas as pl
from jax.experimental.pallas import tpu as pltpu
from jax.experimental.pallas import tpu_sc as plsc

_BANK = 100000
_DIM = 128
_BATCH = 16384

_CHUNK = 4000
_NCHUNK = _BANK // _CHUNK
_FSUB = 400
_NSUB = _CHUNK // _FSUB
_WSPLIT = _BATCH // _CHUNK
_SOFF = _BATCH - _WSPLIT * _CHUNK


def _body(feat_hbm, bank_hbm, lab_hbm, lbank_hbm, out_fb, out_lb,
          fbuf, lbuf, sin0, sin1, sout0, sout1):
    wid = lax.axis_index("s") * 2 + lax.axis_index("c")
    base = wid * _CHUNK
    sin = (sin0, sin1)
    sout = (sout0, sout1)

    def _pipeline(src, first_sub):
        def in_copy(j):
            b = j % 2
            row0 = base + j * _FSUB
            return pltpu.make_async_copy(
                src.at[pl.ds(row0, _FSUB), :], fbuf.at[b], sin[b])

        def out_copy(j):
            b = j % 2
            row0 = base + j * _FSUB
            return pltpu.make_async_copy(
                fbuf.at[b], out_fb.at[pl.ds(row0, _FSUB), :], sout[b])

        in_copy(first_sub).start()
        for j in range(first_sub, _NSUB):
            if j + 1 < _NSUB:
                if j - 1 >= first_sub:
                    out_copy(j - 1).wait()
                in_copy(j + 1).start()
            in_copy(j).wait()
            out_copy(j).start()
        if _NSUB - 2 >= first_sub:
            out_copy(_NSUB - 2).wait()
        out_copy(_NSUB - 1).wait()

    @pl.when(wid < _WSPLIT)
    def _():
        _pipeline(feat_hbm, 0)
        pltpu.sync_copy(lab_hbm.at[pl.ds(base, _CHUNK)], lbuf)
        pltpu.sync_copy(lbuf, out_lb.at[pl.ds(base, _CHUNK)])

    @pl.when(wid == _WSPLIT)
    def _():
        pltpu.sync_copy(feat_hbm.at[pl.ds(base, _SOFF), :],
                        fbuf.at[0, pl.ds(0, _SOFF), :])
        pltpu.sync_copy(bank_hbm.at[pl.ds(base + _SOFF, _FSUB - _SOFF), :],
                        fbuf.at[0, pl.ds(_SOFF, _FSUB - _SOFF), :])
        pltpu.sync_copy(fbuf.at[0], out_fb.at[pl.ds(base, _FSUB), :])
        _pipeline(bank_hbm, 1)
        pltpu.sync_copy(lab_hbm.at[pl.ds(base, _SOFF)], lbuf.at[pl.ds(0, _SOFF)])
        pltpu.sync_copy(lbank_hbm.at[pl.ds(base + _SOFF, _CHUNK - _SOFF)],
                        lbuf.at[pl.ds(_SOFF, _CHUNK - _SOFF)])
        pltpu.sync_copy(lbuf, out_lb.at[pl.ds(base, _CHUNK)])

    @pl.when(jnp.logical_and(wid > _WSPLIT, wid < _NCHUNK))
    def _():
        _pipeline(bank_hbm, 0)
        pltpu.sync_copy(lbank_hbm.at[pl.ds(base, _CHUNK)], lbuf)
        pltpu.sync_copy(lbuf, out_lb.at[pl.ds(base, _CHUNK)])


def kernel(features, labels, feature_bank, label_bank):
    run = functools.partial(
        pl.kernel,
        out_type=[
            jax.ShapeDtypeStruct((_BANK, _DIM), feature_bank.dtype),
            jax.ShapeDtypeStruct((_BANK,), label_bank.dtype),
        ],
        mesh=plsc.VectorSubcoreMesh(core_axis_name="c", subcore_axis_name="s"),
        scratch_types=[
            pltpu.VMEM((2, _FSUB, _DIM), feature_bank.dtype),
            pltpu.VMEM((_CHUNK,), label_bank.dtype),
            pltpu.SemaphoreType.DMA,
            pltpu.SemaphoreType.DMA,
            pltpu.SemaphoreType.DMA,
            pltpu.SemaphoreType.DMA,
        ],
    )(_body)
    out_fb, out_lb = run(features, feature_bank, labels, label_bank)
    return out_fb, out_lb


# speedup vs baseline: 1.3415x; 1.2316x over previous
import functools

import jax
import jax.numpy as jnp
from jax import lax
from jax.experimental import pallas as pl
from jax.experimental.pallas import tpu as pltpu
from jax.experimental.pallas import tpu_sc as plsc

_BANK = 100000
_DIM = 128
_BATCH = 16384

_BLK = 8192
_NB = (_BANK + _BLK - 1) // _BLK
_SPLIT = _BATCH // _BLK


def _fb_body(feat_ref, bank_ref, out_fb_ref):
    i = pl.program_id(0)

    @pl.when(i < _SPLIT)
    def _():
        out_fb_ref[...] = feat_ref[...]

    @pl.when(i >= _SPLIT)
    def _():
        out_fb_ref[...] = bank_ref[...]


def _fb_copy(features, feature_bank):
    return pl.pallas_call(
        _fb_body,
        grid=(_NB,),
        in_specs=[
            pl.BlockSpec((_BLK, _DIM), lambda i: (jnp.minimum(i, _SPLIT - 1), 0)),
            pl.BlockSpec((_BLK, _DIM), lambda i: (jnp.maximum(i, _SPLIT), 0)),
        ],
        out_specs=pl.BlockSpec((_BLK, _DIM), lambda i: (i, 0)),
        out_shape=jax.ShapeDtypeStruct((_BANK, _DIM), feature_bank.dtype),
    )(features, feature_bank)


_LCHUNK = 4000
_NCHUNK = _BANK // _LCHUNK
_LSPLIT = _BATCH // _LCHUNK
_LOFF = _BATCH - _LSPLIT * _LCHUNK


def _lb_body(lab_hbm, lbank_hbm, out_hbm, buf):
    wid = lax.axis_index("s") * 2 + lax.axis_index("c")
    base = wid * _LCHUNK

    @pl.when(wid < _LSPLIT)
    def _():
        pltpu.sync_copy(lab_hbm.at[pl.ds(base, _LCHUNK)], buf)
        pltpu.sync_copy(buf, out_hbm.at[pl.ds(base, _LCHUNK)])

    @pl.when(wid == _LSPLIT)
    def _():
        pltpu.sync_copy(lab_hbm.at[pl.ds(base, _LOFF)], buf.at[pl.ds(0, _LOFF)])
        pltpu.sync_copy(lbank_hbm.at[pl.ds(base + _LOFF, _LCHUNK - _LOFF)],
                        buf.at[pl.ds(_LOFF, _LCHUNK - _LOFF)])
        pltpu.sync_copy(buf, out_hbm.at[pl.ds(base, _LCHUNK)])

    @pl.when(jnp.logical_and(wid > _LSPLIT, wid < _NCHUNK))
    def _():
        pltpu.sync_copy(lbank_hbm.at[pl.ds(base, _LCHUNK)], buf)
        pltpu.sync_copy(buf, out_hbm.at[pl.ds(base, _LCHUNK)])


def _lb_copy(labels, label_bank):
    run = functools.partial(
        pl.kernel,
        out_type=jax.ShapeDtypeStruct((_BANK,), label_bank.dtype),
        mesh=plsc.VectorSubcoreMesh(core_axis_name="c", subcore_axis_name="s"),
        scratch_types=[pltpu.VMEM((_LCHUNK,), label_bank.dtype)],
    )(_lb_body)
    return run(labels, label_bank)


def kernel(features, labels, feature_bank, label_bank):
    out_lb = _lb_copy(labels, label_bank)
    out_fb = _fb_copy(features, feature_bank)
    return out_fb, out_lb
